# baseline (speedup 1.0000x reference)
import functools

import jax
import jax.numpy as jnp
from jax import lax
from jax.experimental import pallas as pl
from jax.experimental.pallas import tpu as pltpu

C = 96
C2 = 192
K = 9
G = 4
N = 1024
BIG_I = 1 << 30


def _grapher_body(x_ref, w1_ref, b1_ref, g1_ref, beta1_ref, m1p_ref, v1_ref,
                  wa_ref, ba_ref, wd_ref, t2_ref,
                  w2a_ref, w2b_ref, b2_ref, out_ref):
    xb = x_ref[0]
    w1 = w1_ref[...]
    b1 = b1_ref[...]
    h = lax.dot_general(xb, w1, (((1,), (1,)), ((), ()))) + b1
    feat = ((h - m1p_ref[...]) / jnp.sqrt(v1_ref[...] + 1e-5)
            * g1_ref[...] + beta1_ref[...])

    norm = jnp.sqrt(jnp.sum(feat * feat, axis=1, keepdims=True))
    fn = feat / jnp.maximum(norm, 1e-12)
    xsq = jnp.sum(fn * fn, axis=1, keepdims=True)
    gram = lax.dot_general(fn, fn, (((1,), (1,)), ((), ())))
    dist = xsq - 2.0 * gram + jnp.reshape(xsq, (1, N))

    m1 = jnp.maximum(
        lax.dot_general(feat, wa_ref[...], (((1,), (1,)), ((), ()))) + ba_ref[...],
        0.0)

    zd = lax.dot_general(feat, wd_ref[...], (((1,), (1,)), ((), ())))
    zdm = zd - t2_ref[...]

    zd_aug = jnp.concatenate([zd, jnp.ones((N, 32), jnp.float32)],
                             axis=1).astype(jnp.bfloat16)

    d = dist
    m2 = jnp.full((N, C), -jnp.inf, jnp.float32)
    mn = jnp.min(d, axis=1, keepdims=True)
    for k in range(K):
        sel = d == mn
        onehot = sel.astype(jnp.bfloat16)
        zj = lax.dot_general(onehot, zd_aug, (((1,), (0,)), ((), ())),
                             preferred_element_type=jnp.float32)
        m2 = jnp.maximum(m2, zj[:, :C] / zj[:, C:C + 1] - zdm)
        if k < K - 1:
            d = jnp.where(sel, jnp.float32(jnp.inf), d)
            mn = jnp.min(d, axis=1, keepdims=True)
    m2 = jnp.maximum(m2, 0.0)

    out = (lax.dot_general(m1, w2a_ref[...], (((1,), (1,)), ((), ())))
           + lax.dot_general(m2, w2b_ref[...], (((1,), (1,)), ((), ())))
           + b2_ref[...] + xb)
    out_ref[0] = out


@jax.jit
def kernel(x, fc1_w, fc1_b, fc1_g, fc1_beta, fc1_m, fc1_v,
           gc_w, gc_b, gc_g, gc_beta, gc_m, gc_v,
           fc2_w, fc2_b, fc2_g, fc2_beta, fc2_m, fc2_v):
    B = x.shape[0]
    H, W = x.shape[2], x.shape[3]
    eps = 1e-5

    w1 = fc1_w
    b1 = fc1_b[None, :]

    wg = gc_w.reshape(G, C2 // G, C2 // G)
    z48 = jnp.zeros((C2 // G, C2 // G), jnp.float32)
    bd01 = jnp.block([[wg[0], z48], [z48, wg[1]]])
    bd23 = jnp.block([[wg[2], z48], [z48, wg[3]]])
    sa = gc_g[:C] / jnp.sqrt(gc_v[:C] + eps)
    wa = sa[:, None] * bd01
    ba = (sa * (gc_b[:C] - gc_m[:C]) + gc_beta[:C])[None, :]
    s2 = gc_g[C:] / jnp.sqrt(gc_v[C:] + eps)
    wd = s2[:, None] * bd23
    t2 = (s2 * (gc_b[C:] - gc_m[C:]) + gc_beta[C:])[None, :]

    s3 = fc2_g / jnp.sqrt(fc2_v + eps)
    w2 = s3[:, None] * fc2_w
    w2a, w2b = w2[:, :C], w2[:, C:]
    b2 = (s3 * (fc2_b - fc2_m) + fc2_beta)[None, :]

    xt = x.reshape(B, C, N).transpose(0, 2, 1)

    rep2 = lambda b: (0, 0)
    out = pl.pallas_call(
        _grapher_body,
        grid=(B,),
        in_specs=[
            pl.BlockSpec((1, N, C), lambda b: (b, 0, 0)),
            pl.BlockSpec((C, C), rep2),
            pl.BlockSpec((1, C), rep2),
            pl.BlockSpec((1, C), rep2),
            pl.BlockSpec((1, C), rep2),
            pl.BlockSpec((1, C), rep2),
            pl.BlockSpec((1, C), rep2),
            pl.BlockSpec((C, C), rep2),
            pl.BlockSpec((1, C), rep2),
            pl.BlockSpec((C, C), rep2),
            pl.BlockSpec((1, C), rep2),
            pl.BlockSpec((C, C), rep2),
            pl.BlockSpec((C, C), rep2),
            pl.BlockSpec((1, C), rep2),
        ],
        out_specs=pl.BlockSpec((1, N, C), lambda b: (b, 0, 0)),
        out_shape=jax.ShapeDtypeStruct((B, N, C), jnp.float32),
        compiler_params=pltpu.CompilerParams(
            dimension_semantics=("parallel",),
        ),
    )(xt, w1, b1, fc1_g[None, :], fc1_beta[None, :], fc1_m[None, :],
      fc1_v[None, :], wa, ba, wd, t2, w2a, w2b, b2)

    return out.transpose(0, 2, 1).reshape(B, C, H, W)


# speedup vs baseline: 1.1809x; 1.1809x over previous
import functools

import jax
import jax.numpy as jnp
from jax import lax
from jax.experimental import pallas as pl
from jax.experimental.pallas import tpu as pltpu

C = 96
C2 = 192
K = 9
G = 4
N = 1024
BIG_I = 1 << 30


def _grapher_body(x_ref, w1_ref, b1_ref, g1_ref, beta1_ref, m1p_ref, v1_ref,
                  wa_ref, ba_ref, wd_ref, t2_ref,
                  w2a_ref, w2b_ref, b2_ref, out_ref):
    xb = x_ref[0]
    w1 = w1_ref[...]
    b1 = b1_ref[...]
    h = lax.dot_general(xb, w1, (((1,), (1,)), ((), ()))) + b1
    feat = ((h - m1p_ref[...]) / jnp.sqrt(v1_ref[...] + 1e-5)
            * g1_ref[...] + beta1_ref[...])

    norm = jnp.sqrt(jnp.sum(feat * feat, axis=1, keepdims=True))
    fn = feat / jnp.maximum(norm, 1e-12)
    xsq = jnp.sum(fn * fn, axis=1, keepdims=True)
    gram = lax.dot_general(fn, fn, (((1,), (1,)), ((), ())))
    eye = (lax.broadcasted_iota(jnp.int32, (N, N), 0)
           == lax.broadcasted_iota(jnp.int32, (N, N), 1))
    dist = jnp.where(eye, jnp.float32(jnp.inf),
                     xsq - 2.0 * gram + jnp.reshape(xsq, (1, N)))

    m1 = jnp.maximum(
        lax.dot_general(feat, wa_ref[...], (((1,), (1,)), ((), ()))) + ba_ref[...],
        0.0)

    zd = lax.dot_general(feat, wd_ref[...], (((1,), (1,)), ((), ())))
    zdm = zd - t2_ref[...]

    zd_aug = jnp.concatenate([zd, jnp.ones((N, 32), jnp.float32)], axis=1)

    d = dist
    m2 = zd - zdm
    mn = jnp.min(d, axis=1, keepdims=True)
    for k in range(K - 1):
        sel = d == mn
        onehot = sel.astype(jnp.float32)
        zj = lax.dot_general(onehot, zd_aug, (((1,), (0,)), ((), ())))
        m2 = jnp.maximum(m2, zj[:, :C] / zj[:, C:C + 1] - zdm)
        if k < K - 2:
            d = jnp.where(sel, jnp.float32(jnp.inf), d)
            mn = jnp.min(d, axis=1, keepdims=True)
    m2 = jnp.maximum(m2, 0.0)

    out = (lax.dot_general(m1, w2a_ref[...], (((1,), (1,)), ((), ())))
           + lax.dot_general(m2, w2b_ref[...], (((1,), (1,)), ((), ())))
           + b2_ref[...] + xb)
    out_ref[0] = out


@jax.jit
def kernel(x, fc1_w, fc1_b, fc1_g, fc1_beta, fc1_m, fc1_v,
           gc_w, gc_b, gc_g, gc_beta, gc_m, gc_v,
           fc2_w, fc2_b, fc2_g, fc2_beta, fc2_m, fc2_v):
    B = x.shape[0]
    H, W = x.shape[2], x.shape[3]
    eps = 1e-5

    w1 = fc1_w
    b1 = fc1_b[None, :]

    wg = gc_w.reshape(G, C2 // G, C2 // G)
    z48 = jnp.zeros((C2 // G, C2 // G), jnp.float32)
    bd01 = jnp.block([[wg[0], z48], [z48, wg[1]]])
    bd23 = jnp.block([[wg[2], z48], [z48, wg[3]]])
    sa = gc_g[:C] / jnp.sqrt(gc_v[:C] + eps)
    wa = sa[:, None] * bd01
    ba = (sa * (gc_b[:C] - gc_m[:C]) + gc_beta[:C])[None, :]
    s2 = gc_g[C:] / jnp.sqrt(gc_v[C:] + eps)
    wd = s2[:, None] * bd23
    t2 = (s2 * (gc_b[C:] - gc_m[C:]) + gc_beta[C:])[None, :]

    s3 = fc2_g / jnp.sqrt(fc2_v + eps)
    w2 = s3[:, None] * fc2_w
    w2a, w2b = w2[:, :C], w2[:, C:]
    b2 = (s3 * (fc2_b - fc2_m) + fc2_beta)[None, :]

    xt = x.reshape(B, C, N).transpose(0, 2, 1)

    rep2 = lambda b: (0, 0)
    out = pl.pallas_call(
        _grapher_body,
        grid=(B,),
        in_specs=[
            pl.BlockSpec((1, N, C), lambda b: (b, 0, 0)),
            pl.BlockSpec((C, C), rep2),
            pl.BlockSpec((1, C), rep2),
            pl.BlockSpec((1, C), rep2),
            pl.BlockSpec((1, C), rep2),
            pl.BlockSpec((1, C), rep2),
            pl.BlockSpec((1, C), rep2),
            pl.BlockSpec((C, C), rep2),
            pl.BlockSpec((1, C), rep2),
            pl.BlockSpec((C, C), rep2),
            pl.BlockSpec((1, C), rep2),
            pl.BlockSpec((C, C), rep2),
            pl.BlockSpec((C, C), rep2),
            pl.BlockSpec((1, C), rep2),
        ],
        out_specs=pl.BlockSpec((1, N, C), lambda b: (b, 0, 0)),
        out_shape=jax.ShapeDtypeStruct((B, N, C), jnp.float32),
        compiler_params=pltpu.CompilerParams(
            dimension_semantics=("parallel",),
        ),
    )(xt, w1, b1, fc1_g[None, :], fc1_beta[None, :], fc1_m[None, :],
      fc1_v[None, :], wa, ba, wd, t2, w2a, w2b, b2)

    return out.transpose(0, 2, 1).reshape(B, C, H, W)


# speedup vs baseline: 1.1857x; 1.0041x over previous
import functools

import jax
import jax.numpy as jnp
from jax import lax
from jax.experimental import pallas as pl
from jax.experimental.pallas import tpu as pltpu

C = 96
C2 = 192
K = 9
G = 4
N = 1024
BIG_I = 1 << 30


def _grapher_body(x_ref, w1_ref, b1_ref, g1_ref, beta1_ref, m1p_ref, v1_ref,
                  wa_ref, ba_ref, wd_ref, t2_ref,
                  w2a_ref, w2b_ref, b2_ref, out_ref):
    xb = x_ref[0]
    w1 = w1_ref[...]
    b1 = b1_ref[...]
    h = lax.dot_general(xb, w1, (((1,), (1,)), ((), ()))) + b1
    feat = ((h - m1p_ref[...]) / jnp.sqrt(v1_ref[...] + 1e-5)
            * g1_ref[...] + beta1_ref[...])

    norm = jnp.sqrt(jnp.sum(feat * feat, axis=1, keepdims=True))
    fn = feat / jnp.maximum(norm, 1e-12)
    xsq = jnp.sum(fn * fn, axis=1, keepdims=True)
    gram = lax.dot_general(fn, fn, (((1,), (1,)), ((), ())))
    eye = (lax.broadcasted_iota(jnp.int32, (N, N), 0)
           == lax.broadcasted_iota(jnp.int32, (N, N), 1))
    dist = jnp.where(eye, jnp.float32(jnp.inf),
                     xsq - 2.0 * gram + jnp.reshape(xsq, (1, N)))

    m1 = jnp.maximum(
        lax.dot_general(feat, wa_ref[...], (((1,), (1,)), ((), ()))) + ba_ref[...],
        0.0)

    zd = lax.dot_general(feat, wd_ref[...], (((1,), (1,)), ((), ())))
    zdm = zd - t2_ref[...]

    zd_aug = jnp.concatenate([zd, jnp.ones((N, 32), jnp.float32)], axis=1)

    d = dist
    m2 = zd - zdm
    mn = jnp.min(d, axis=1, keepdims=True)
    for k in range(K - 1):
        sel = d == mn
        onehot = sel.astype(jnp.float32)
        zj = lax.dot_general(onehot, zd_aug, (((1,), (0,)), ((), ())))
        rc = 1.0 / zj[:, C:C + 1]
        m2 = jnp.maximum(m2, zj[:, :C] * rc - zdm)
        if k < K - 2:
            mn = jnp.min(jnp.where(d > mn, d, jnp.float32(jnp.inf)),
                         axis=1, keepdims=True)
    m2 = jnp.maximum(m2, 0.0)

    out = (lax.dot_general(m1, w2a_ref[...], (((1,), (1,)), ((), ())))
           + lax.dot_general(m2, w2b_ref[...], (((1,), (1,)), ((), ())))
           + b2_ref[...] + xb)
    out_ref[0] = out


@jax.jit
def kernel(x, fc1_w, fc1_b, fc1_g, fc1_beta, fc1_m, fc1_v,
           gc_w, gc_b, gc_g, gc_beta, gc_m, gc_v,
           fc2_w, fc2_b, fc2_g, fc2_beta, fc2_m, fc2_v):
    B = x.shape[0]
    H, W = x.shape[2], x.shape[3]
    eps = 1e-5

    w1 = fc1_w
    b1 = fc1_b[None, :]

    wg = gc_w.reshape(G, C2 // G, C2 // G)
    z48 = jnp.zeros((C2 // G, C2 // G), jnp.float32)
    bd01 = jnp.block([[wg[0], z48], [z48, wg[1]]])
    bd23 = jnp.block([[wg[2], z48], [z48, wg[3]]])
    sa = gc_g[:C] / jnp.sqrt(gc_v[:C] + eps)
    wa = sa[:, None] * bd01
    ba = (sa * (gc_b[:C] - gc_m[:C]) + gc_beta[:C])[None, :]
    s2 = gc_g[C:] / jnp.sqrt(gc_v[C:] + eps)
    wd = s2[:, None] * bd23
    t2 = (s2 * (gc_b[C:] - gc_m[C:]) + gc_beta[C:])[None, :]

    s3 = fc2_g / jnp.sqrt(fc2_v + eps)
    w2 = s3[:, None] * fc2_w
    w2a, w2b = w2[:, :C], w2[:, C:]
    b2 = (s3 * (fc2_b - fc2_m) + fc2_beta)[None, :]

    xt = x.reshape(B, C, N).transpose(0, 2, 1)

    rep2 = lambda b: (0, 0)
    out = pl.pallas_call(
        _grapher_body,
        grid=(B,),
        in_specs=[
            pl.BlockSpec((1, N, C), lambda b: (b, 0, 0)),
            pl.BlockSpec((C, C), rep2),
            pl.BlockSpec((1, C), rep2),
            pl.BlockSpec((1, C), rep2),
            pl.BlockSpec((1, C), rep2),
            pl.BlockSpec((1, C), rep2),
            pl.BlockSpec((1, C), rep2),
            pl.BlockSpec((C, C), rep2),
            pl.BlockSpec((1, C), rep2),
            pl.BlockSpec((C, C), rep2),
            pl.BlockSpec((1, C), rep2),
            pl.BlockSpec((C, C), rep2),
            pl.BlockSpec((C, C), rep2),
            pl.BlockSpec((1, C), rep2),
        ],
        out_specs=pl.BlockSpec((1, N, C), lambda b: (b, 0, 0)),
        out_shape=jax.ShapeDtypeStruct((B, N, C), jnp.float32),
        compiler_params=pltpu.CompilerParams(
            dimension_semantics=("parallel",),
        ),
    )(xt, w1, b1, fc1_g[None, :], fc1_beta[None, :], fc1_m[None, :],
      fc1_v[None, :], wa, ba, wd, t2, w2a, w2b, b2)

    return out.transpose(0, 2, 1).reshape(B, C, H, W)
